# baseline (speedup 1.0000x reference)
import functools

import jax
import jax.numpy as jnp
from jax import lax
from jax.experimental import pallas as pl
from jax.experimental.pallas import tpu as pltpu
from jax.experimental.pallas import tpu_sc as plsc

_NUM_WORKERS = 32
_ROWS = 8


@functools.lru_cache(maxsize=None)
def _make_gather(N, T, V, D):
    rows_per_w = N // _NUM_WORKERS
    n_chunk = rows_per_w // _ROWS
    mesh = plsc.VectorSubcoreMesh(core_axis_name="c", subcore_axis_name="s")

    @functools.partial(
        pl.kernel,
        mesh=mesh,
        out_type=jax.ShapeDtypeStruct((N, T, D), jnp.float32),
        scratch_types=[
            pltpu.VMEM((_ROWS, T), jnp.int32),
            pltpu.VMEM((_ROWS, T, D), jnp.float32),
            pltpu.SemaphoreType.DMA,
        ],
        compiler_params=pltpu.CompilerParams(use_tc_tiling_on_sc=False),
    )
    def gather_kernel(idx_hbm, table_hbm, out_hbm, idx_v, rows_v, sem):
        wid = lax.axis_index("s") * 2 + lax.axis_index("c")
        base = wid * rows_per_w

        def body(i, carry):
            row0 = base + i * _ROWS
            pltpu.sync_copy(idx_hbm.at[pl.ds(row0, _ROWS), :], idx_v)
            for j in range(_ROWS):
                pltpu.async_copy(table_hbm.at[idx_v.at[j]], rows_v.at[j], sem)
            for j in range(_ROWS):
                pltpu.make_async_copy(table_hbm.at[idx_v.at[j]], rows_v.at[j],
                                      sem).wait()
            pltpu.sync_copy(rows_v, out_hbm.at[pl.ds(row0, _ROWS), :, :])
            return carry

        lax.fori_loop(0, n_chunk, body, 0)

    return gather_kernel


def kernel(token_ids, embedding):
    V, D = embedding.shape
    N, T = token_ids.shape
    idx = token_ids.astype(jnp.int32)
    return _make_gather(N, T, V, D)(idx, embedding)


# speedup vs baseline: 1.0580x; 1.0580x over previous
import functools

import jax
import jax.numpy as jnp
from jax import lax
from jax.experimental import pallas as pl
from jax.experimental.pallas import tpu as pltpu
from jax.experimental.pallas import tpu_sc as plsc

_NUM_WORKERS = 32


@functools.lru_cache(maxsize=None)
def _make_gather(N, T, V, D):
    TB = T // 8
    NB = N // 128
    blocks_per_w = (TB * NB) // _NUM_WORKERS
    mesh = plsc.VectorSubcoreMesh(core_axis_name="c", subcore_axis_name="s")

    @functools.partial(
        pl.kernel,
        mesh=mesh,
        out_type=jax.ShapeDtypeStruct((T, N, D), jnp.float32),
        scratch_types=[
            pltpu.VMEM((8, 128), jnp.int32),
            pltpu.VMEM((1024, D), jnp.float32),
            pltpu.SemaphoreType.DMA,
        ],
        compiler_params=pltpu.CompilerParams(use_tc_tiling_on_sc=False),
    )
    def gather_kernel(idx_hbm, table_hbm, out_hbm, idx_v, rows_v, sem):
        wid = lax.axis_index("s") * 2 + lax.axis_index("c")
        base = wid * blocks_per_w

        def block_body(b, carry):
            blk = base + b
            tb = blk // NB
            nb = blk % NB
            n0 = nb * 128
            pltpu.sync_copy(idx_hbm.at[tb, nb], idx_v)
            for tr in range(8):
                pltpu.async_copy(table_hbm.at[idx_v.at[tr]],
                                 rows_v.at[pl.ds(tr * 128, 128)], sem)
            for tr in range(8):
                pltpu.make_async_copy(table_hbm.at[idx_v.at[tr]],
                                      rows_v.at[pl.ds(tr * 128, 128)],
                                      sem).wait()
            for tr in range(8):
                pltpu.sync_copy(rows_v.at[pl.ds(tr * 128, 128)],
                                out_hbm.at[tb * 8 + tr, pl.ds(n0, 128), :])
            return carry

        lax.fori_loop(0, blocks_per_w, block_body, 0)

    return gather_kernel


def kernel(token_ids, embedding):
    V, D = embedding.shape
    N, T = token_ids.shape
    idx_tiles = (token_ids.astype(jnp.int32).T
                 .reshape(T // 8, 8, N // 128, 128)
                 .transpose(0, 2, 1, 3))
    x_t = _make_gather(N, T, V, D)(idx_tiles, embedding)
    return x_t.transpose(1, 0, 2)
